# Initial kernel scaffold; baseline (speedup 1.0000x reference)
import jax
import jax.numpy as jnp
from jax.experimental import pallas as pl


def kernel(node_features, edge_index, edge_features, W1, att_src1, att_dst1, b1, W2, att_src2, att_dst2, b2, We, be, Wc, bc, Wmu, bmu, Wlv, blv):
    raise NotImplementedError("write your pallas kernel here")


# speedup vs baseline: 1.1608x; 1.1608x over previous
import jax
import jax.numpy as jnp
from jax.experimental import pallas as pl


def _mm_body(x_ref, w_ref, o_ref):
    o_ref[...] = jnp.dot(x_ref[...], w_ref[...],
                         preferred_element_type=jnp.float32)


def _matmul(x, w, bm=1000):
    m, k = x.shape
    _, n = w.shape
    return pl.pallas_call(
        _mm_body,
        grid=(m // bm,),
        in_specs=[
            pl.BlockSpec((bm, k), lambda i: (i, 0)),
            pl.BlockSpec((k, n), lambda i: (0, 0)),
        ],
        out_specs=pl.BlockSpec((bm, n), lambda i: (i, 0)),
        out_shape=jax.ShapeDtypeStruct((m, n), jnp.float32),
    )(x, w)


def _gat(x, src, dst, W, att_src, att_dst, b):
    n = x.shape[0]
    h = _matmul(x, W)
    alpha_src = (h * att_src[None, :]).sum(-1)
    alpha_dst = (h * att_dst[None, :]).sum(-1)
    e = alpha_src[src] + alpha_dst[dst]
    e = jax.nn.leaky_relu(e, negative_slope=0.2)
    emax = jax.ops.segment_max(e, dst, num_segments=n)
    emax = jnp.where(jnp.isfinite(emax), emax, 0.0)
    ex = jnp.exp(e - emax[dst])
    denom = jax.ops.segment_sum(ex, dst, num_segments=n)
    alpha = ex / (denom[dst] + 1e-16)
    out = jax.ops.segment_sum(alpha[:, None] * h[src], dst, num_segments=n)
    return out + b[None, :]


def kernel(node_features, edge_index, edge_features, W1, att_src1, att_dst1,
           b1, W2, att_src2, att_dst2, b2, We, be, Wc, bc, Wmu, bmu, Wlv,
           blv):
    src = edge_index[0]
    dst = edge_index[1]
    x = _gat(node_features, src, dst, W1, att_src1, att_dst1, b1)
    x = jax.nn.relu(x)
    x = _gat(x, src, dst, W2, att_src2, att_dst2, b2)
    x = jax.nn.relu(x)
    edge_emb = jax.nn.relu(_matmul(edge_features, We, bm=1000) + be[None, :])
    graph_emb = jnp.concatenate([
        jnp.mean(x, axis=0, keepdims=True),
        jnp.mean(edge_emb, axis=0, keepdims=True),
    ], axis=1)
    graph_emb = jax.nn.relu(graph_emb @ Wc + bc[None, :])
    mu = graph_emb @ Wmu + bmu[None, :]
    log_var = graph_emb @ Wlv + blv[None, :]
    return (mu, log_var)


# speedup vs baseline: 5.7344x; 4.9403x over previous
import functools

import jax
import jax.numpy as jnp
from jax import lax
from jax.experimental import pallas as pl
from jax.experimental.pallas import tpu as pltpu
from jax.experimental.pallas import tpu_sc as plsc

_HIGH = jax.lax.Precision.HIGHEST


def _tc_embed_body(x_ref, w_ref, asrc_ref, adst_ref, h_ref, aa_ref):
    h = jnp.dot(x_ref[...], w_ref[...], preferred_element_type=jnp.float32,
                precision=_HIGH)
    h_ref[...] = h
    a_s = jnp.sum(h * asrc_ref[...], axis=1)
    a_d = jnp.sum(h * adst_ref[...], axis=1)
    aa_ref[...] = jnp.concatenate(
        [a_s[None, :], a_d[None, :],
         jnp.zeros((14, a_s.shape[0]), jnp.float32)], axis=0)


def _tc_embed(x, w, asrc, adst, bm=640):
    np_, d = x.shape
    hd = w.shape[1]
    return pl.pallas_call(
        _tc_embed_body,
        grid=(np_ // bm,),
        in_specs=[pl.BlockSpec((bm, d), lambda i: (i, 0)),
                  pl.BlockSpec((d, hd), lambda i: (0, 0)),
                  pl.BlockSpec((1, d), lambda i: (0, 0)),
                  pl.BlockSpec((1, d), lambda i: (0, 0))],
        out_specs=[pl.BlockSpec((bm, hd), lambda i: (i, 0)),
                   pl.BlockSpec((16, bm), lambda i: (0, i))],
        out_shape=[jax.ShapeDtypeStruct((np_, hd), jnp.float32),
                   jax.ShapeDtypeStruct((16, np_), jnp.float32)],
    )(x, w, asrc.reshape(1, -1), adst.reshape(1, -1))


def _tc_comb_embed_body(p_ref, b_ref, w_ref, asrc_ref, adst_ref,
                        h_ref, aa_ref, *, bm, n_valid):
    i = pl.program_id(0)
    x = jnp.maximum(p_ref[...] + b_ref[...], 0.0)
    rows = i * bm + lax.broadcasted_iota(jnp.int32, (bm, 1), 0)
    x = jnp.where(rows < n_valid, x, 0.0)
    h = jnp.dot(x, w_ref[...], preferred_element_type=jnp.float32,
                precision=_HIGH)
    h_ref[...] = h
    a_s = jnp.sum(h * asrc_ref[...], axis=1)
    a_d = jnp.sum(h * adst_ref[...], axis=1)
    aa_ref[...] = jnp.concatenate(
        [a_s[None, :], a_d[None, :],
         jnp.zeros((14, a_s.shape[0]), jnp.float32)], axis=0)


def _tc_comb_embed(p, b, w, asrc, adst, n_valid, bm=640):
    np_, hd0 = p.shape
    hd = w.shape[1]
    body = functools.partial(_tc_comb_embed_body, bm=bm, n_valid=n_valid)
    return pl.pallas_call(
        body,
        grid=(np_ // bm,),
        in_specs=[pl.BlockSpec((bm, hd0), lambda i: (i, 0)),
                  pl.BlockSpec((1, hd0), lambda i: (0, 0)),
                  pl.BlockSpec((hd0, hd), lambda i: (0, 0)),
                  pl.BlockSpec((1, hd0), lambda i: (0, 0)),
                  pl.BlockSpec((1, hd0), lambda i: (0, 0))],
        out_specs=[pl.BlockSpec((bm, hd), lambda i: (i, 0)),
                   pl.BlockSpec((16, bm), lambda i: (0, i))],
        out_shape=[jax.ShapeDtypeStruct((np_, hd), jnp.float32),
                   jax.ShapeDtypeStruct((16, np_), jnp.float32)],
    )(p, b.reshape(1, -1), w, asrc.reshape(1, -1), adst.reshape(1, -1))


def _tc_xsum_body(p_ref, b_ref, o_ref, *, bm, n_valid):
    i = pl.program_id(0)
    x = jnp.maximum(p_ref[...] + b_ref[...], 0.0)
    rows = i * bm + lax.broadcasted_iota(jnp.int32, (bm, 1), 0)
    x = jnp.where(rows < n_valid, x, 0.0)
    part = jnp.sum(x.reshape(bm // 8, 8, x.shape[1]), axis=0)

    @pl.when(i == 0)
    def _():
        o_ref[...] = part

    @pl.when(i > 0)
    def _():
        o_ref[...] += part


def _tc_xsum(p, b, n_valid, bm=640):
    np_, hd = p.shape
    body = functools.partial(_tc_xsum_body, bm=bm, n_valid=n_valid)
    return pl.pallas_call(
        body,
        grid=(np_ // bm,),
        in_specs=[pl.BlockSpec((bm, hd), lambda i: (i, 0)),
                  pl.BlockSpec((1, hd), lambda i: (0, 0))],
        out_specs=pl.BlockSpec((8, hd), lambda i: (0, 0)),
        out_shape=jax.ShapeDtypeStruct((8, hd), jnp.float32),
    )(p, b.reshape(1, -1))


def _tc_esum_body(ef_ref, we_ref, be_ref, o_ref, *, bm):
    i = pl.program_id(0)
    emb = jnp.dot(ef_ref[...], we_ref[...], preferred_element_type=jnp.float32,
                  precision=_HIGH)
    emb = jnp.maximum(emb + be_ref[...], 0.0)
    part = jnp.sum(emb.reshape(bm // 8, 8, emb.shape[1]), axis=0)

    @pl.when(i == 0)
    def _():
        o_ref[...] = part

    @pl.when(i > 0)
    def _():
        o_ref[...] += part


def _tc_esum(ef, we, be, bm=2000):
    e, de = ef.shape
    hd = we.shape[1]
    body = functools.partial(_tc_esum_body, bm=bm)
    return pl.pallas_call(
        body,
        grid=(e // bm,),
        in_specs=[pl.BlockSpec((bm, de), lambda i: (i, 0)),
                  pl.BlockSpec((de, hd), lambda i: (0, 0)),
                  pl.BlockSpec((1, hd), lambda i: (0, 0))],
        out_specs=pl.BlockSpec((8, hd), lambda i: (0, 0)),
        out_shape=jax.ShapeDtypeStruct((8, hd), jnp.float32),
    )(ef, we, be.reshape(1, -1))


def _tc_head_body(xs_ref, es_ref, wc_ref, bc_ref, wmu_ref, bmu_ref,
                  wlv_ref, blv_ref, mu_ref, lv_ref, *, n_nodes, n_edges):
    xmean = jnp.sum(xs_ref[...], axis=0, keepdims=True) * (1.0 / n_nodes)
    emean = jnp.sum(es_ref[...], axis=0, keepdims=True) * (1.0 / n_edges)
    g = jnp.concatenate([xmean, emean], axis=1)
    g8 = jnp.broadcast_to(g, (8, g.shape[1]))
    ge = jnp.dot(g8, wc_ref[...], preferred_element_type=jnp.float32,
                 precision=_HIGH) + bc_ref[...]
    ge = jnp.maximum(ge, 0.0)
    mu = jnp.dot(ge, wmu_ref[...], preferred_element_type=jnp.float32,
                 precision=_HIGH) + bmu_ref[...]
    lv = jnp.dot(ge, wlv_ref[...], preferred_element_type=jnp.float32,
                 precision=_HIGH) + blv_ref[...]
    mu_ref[...] = mu[0:1, :]
    lv_ref[...] = lv[0:1, :]


def _tc_head(xs8, es8, wc, bc, wmu, bmu, wlv, blv, n_nodes, n_edges):
    ld = wmu.shape[1]
    body = functools.partial(_tc_head_body, n_nodes=float(n_nodes),
                             n_edges=float(n_edges))
    return pl.pallas_call(
        body,
        out_shape=[jax.ShapeDtypeStruct((1, ld), jnp.float32),
                   jax.ShapeDtypeStruct((1, ld), jnp.float32)],
    )(xs8, es8, wc, bc.reshape(1, -1), wmu, bmu.reshape(1, -1),
      wlv, blv.reshape(1, -1))


def _gat_sc_body(hp, aa, src2d, dst2d, out,
                 asv, adv, srcv, dstv, denf, tmpz, idxb, exb,
                 rows0, rows1, dsh, osh, sem0, sem1,
                 *, nc, ns, rpt, nsl, np_, nq):
    c = lax.axis_index("c")
    s = lax.axis_index("s")

    pltpu.sync_copy(aa.at[pl.ds(0, np_)], asv)
    pltpu.sync_copy(aa.at[pl.ds(np_, np_)], adv)

    nv = np_ // 16
    zv = jnp.zeros((16,), jnp.float32)
    zr = (nq + 128) // ns

    def _zero1d(ref, n16):
        def body(i, _):
            ref[pl.ds(i * 16, 16)] = zv
            return 0
        lax.fori_loop(0, n16, body, 0)

    def _zero2d(ref, nr):
        def body(i, _):
            r = i // 8
            k = i - r * 8
            ref[r, pl.ds(k * 16, 16)] = zv
            return 0
        lax.fori_loop(0, nr * 8, body, 0)

    def _zero_osh():
        _zero2d(rows0, 128)
        pltpu.sync_copy(rows0, osh.at[pl.ds(s * zr, 128)])
        pltpu.sync_copy(rows0.at[pl.ds(0, zr - 128)],
                        osh.at[pl.ds(s * zr + 128, zr - 128)])

    _zero1d(tmpz, nsl // 16)
    pltpu.sync_copy(tmpz, dsh.at[pl.ds(s * nsl, nsl)])
    plsc.subcore_barrier()

    def _vmax(ref):
        tmpz[pl.ds(0, 16)] = jnp.full((16,), -jnp.inf, jnp.float32)

        def body(i, _):
            tmpz[pl.ds(0, 16)] = jnp.maximum(tmpz[pl.ds(0, 16)],
                                             ref[pl.ds(i * 16, 16)])
            return 0
        lax.fori_loop(0, nv, body, 0)
        return jnp.max(tmpz[pl.ds(0, 16)])

    m = _vmax(asv) + _vmax(adv)
    m = jnp.maximum(m, 0.2 * m)

    def _ex16(sl, r):
        svi = srcv[r, sl]
        dvi = dstv[r, sl]
        av = plsc.load_gather(asv, [svi])
        bv = plsc.load_gather(adv, [dvi])
        e = av + bv
        e = jnp.maximum(e, 0.2 * e)
        return jnp.exp(e - m), dvi

    pltpu.sync_copy(src2d.at[pl.ds(s * rpt, rpt)], srcv)
    pltpu.sync_copy(dst2d.at[pl.ds(s * rpt, rpt)], dstv)

    def rbody(r, _):
        def vbody(k, _):
            sl = pl.ds(k * 16, 16)
            ex, _dvi = _ex16(sl, r)
            exb[0, sl] = ex
            return 0
        lax.fori_loop(0, 8, vbody, 0)
        pltpu.sync_copy(exb.at[0], dsh.at[dstv.at[r]], add=True)
        return 0
    lax.fori_loop(0, rpt, rbody, 0)
    plsc.subcore_barrier()
    pltpu.sync_copy(dsh, denf)

    def _gstart(j, buf, sem):
        pltpu.async_copy(hp.at[srcv.at[j]], buf, sem)

    def _gwait(j, buf, sem):
        pltpu.make_async_copy(hp.at[srcv.at[j]], buf, sem).wait()

    for p in range(2):
        lo = (c * 2 + p) * nq
        _zero_osh()
        plsc.subcore_barrier()

        def _proc(j, buf):
            def cb(k, _):
                sl = pl.ds(k * 16, 16)
                ex, dvi = _ex16(sl, j)
                dn = plsc.load_gather(denf, [dvi])
                exb[0, sl] = ex / (dn + 1e-16)
                dl = dvi - lo
                ok = (dl >= 0) & (dl < nq)
                idxb[0, sl] = jnp.where(ok, dl, nq + (dvi & 63))
                return 0
            lax.fori_loop(0, 8, cb, 0)

            def pb(r, _):
                av = plsc.load_gather(
                    exb, [jnp.zeros((16,), jnp.int32),
                          jnp.full((16,), r, jnp.int32)])
                for q in range(8):
                    sl = pl.ds(q * 16, 16)
                    buf[r, sl] = buf[r, sl] * av
                return 0
            lax.fori_loop(0, 128, pb, 0)
            pltpu.sync_copy(buf, osh.at[idxb.at[0]], add=True)

        _gstart(0, rows0, sem0)

        def pbody(t, _):
            j0 = 2 * t
            _gstart(j0 + 1, rows1, sem1)
            _gwait(j0, rows0, sem0)
            _proc(j0, rows0)
            _gstart(j0 + 2, rows0, sem0)
            _gwait(j0 + 1, rows1, sem1)
            _proc(j0 + 1, rows1)
            return 0
        lax.fori_loop(0, rpt // 2 - 1, pbody, 0)
        _gstart(rpt - 1, rows1, sem1)
        _gwait(rpt - 2, rows0, sem0)
        _proc(rpt - 2, rows0)
        _gwait(rpt - 1, rows1, sem1)
        _proc(rpt - 1, rows1)

        plsc.subcore_barrier()
        nw = nq // ns
        pltpu.sync_copy(osh.at[pl.ds(s * nw, nw)],
                        out.at[pl.ds(lo + s * nw, nw)])
        plsc.subcore_barrier()


def _gat_sc(hp, aa, src2d, dst2d):
    aa = aa.reshape(-1)
    np_, hd = hp.shape
    rows_tot = src2d.shape[0]
    nc, ns = 2, 16
    rpt = rows_tot // ns
    nsl = np_ // ns
    nq = np_ // 4
    mesh = plsc.VectorSubcoreMesh(core_axis_name="c", subcore_axis_name="s",
                                  num_cores=nc, num_subcores=ns)
    body = functools.partial(_gat_sc_body, nc=nc, ns=ns, rpt=rpt, nsl=nsl,
                             np_=np_, nq=nq)
    return pl.kernel(
        body,
        out_type=jax.ShapeDtypeStruct((np_, hd), jnp.float32),
        mesh=mesh,
        compiler_params=pltpu.CompilerParams(needs_layout_passes=False),
        scratch_types=[
            pltpu.VMEM((np_,), jnp.float32),
            pltpu.VMEM((np_,), jnp.float32),
            pltpu.VMEM((rpt, 128), jnp.int32),
            pltpu.VMEM((rpt, 128), jnp.int32),
            pltpu.VMEM((np_,), jnp.float32),
            pltpu.VMEM((nsl,), jnp.float32),
            pltpu.VMEM((8, 128), jnp.int32),
            pltpu.VMEM((8, 128), jnp.float32),
            pltpu.VMEM((128, hd), jnp.float32),
            pltpu.VMEM((128, hd), jnp.float32),
            pltpu.VMEM_SHARED((np_,), jnp.float32),
            pltpu.VMEM_SHARED((np_ // 4 + 128, hd), jnp.float32),
            pltpu.SemaphoreType.DMA,
            pltpu.SemaphoreType.DMA,
        ],
    )(hp, aa, src2d, dst2d)


def kernel(node_features, edge_index, edge_features, W1, att_src1, att_dst1,
           b1, W2, att_src2, att_dst2, b2, We, be, Wc, bc, Wmu, bmu, Wlv,
           blv):
    n, _ = node_features.shape
    e = edge_index.shape[1]

    np_ = ((n + 255) // 256) * 256
    ep = ((e + 32767) // 32768) * 32768
    pad_e = ep - e

    src = edge_index[0]
    dst = edge_index[1]
    srcp = jnp.concatenate(
        [src, jnp.zeros((pad_e,), jnp.int32)]).reshape(ep // 128, 128)
    dstp = jnp.concatenate(
        [dst, jnp.full((pad_e,), n, jnp.int32)]).reshape(ep // 128, 128)
    xp = jnp.pad(node_features, ((0, np_ - n), (0, 0)))

    h1, aa1 = _tc_embed(xp, W1, att_src1, att_dst1)
    p1 = _gat_sc(h1, aa1, srcp, dstp)
    h2, aa2 = _tc_comb_embed(p1, b1, W2, att_src2, att_dst2, n)
    p2 = _gat_sc(h2, aa2, srcp, dstp)
    xs8 = _tc_xsum(p2, b2, n)
    es8 = _tc_esum(edge_features, We, be)
    mu, log_var = _tc_head(xs8, es8, Wc, bc, Wmu, bmu, Wlv, blv, n, e)
    return (mu, log_var)
